# baseline (speedup 1.0000x reference)
import functools

import jax
import jax.numpy as jnp
from jax.experimental import pallas as pl
from jax.experimental.pallas import tpu as pltpu

_DIM = 512
_N_HASHES = 256
_BANDWIDTH = 4.0
_N_BUCKETS = 1024

_CHUNK = 4096
_NBUF = 4


def _lsh_manual(x_hbm, rv_ref, out_hbm, x_buf, out_buf, in_sems, out_sems):
    n = x_hbm.shape[0]
    nchunk = n // _CHUNK

    def in_copy(i, slot):
        return pltpu.make_async_copy(
            x_hbm.at[pl.ds(i * _CHUNK, _CHUNK), :],
            x_buf.at[pl.ds(slot * _CHUNK, _CHUNK), :],
            in_sems.at[slot],
        )

    def out_copy(i, slot):
        return pltpu.make_async_copy(
            out_buf.at[pl.ds(slot * _CHUNK, _CHUNK), :],
            out_hbm.at[pl.ds(i * _CHUNK, _CHUNK), :],
            out_sems.at[slot],
        )

    for b in range(min(_NBUF, nchunk)):
        in_copy(b, b).start()

    for i in range(nchunk):
        slot = i % _NBUF
        in_copy(i, slot).wait()
        if i >= _NBUF:
            out_copy(i - _NBUF, slot).wait()
        xs = x_buf[pl.ds(slot * _CHUNK, _CHUNK), :]
        proj = jnp.dot(xs, rv_ref[...], preferred_element_type=jnp.float32)
        buckets = jnp.floor(proj * (1.0 / _BANDWIDTH)).astype(jnp.int32) & (
            _N_BUCKETS - 1
        )
        out_buf[pl.ds(slot * _CHUNK, _CHUNK), :] = buckets.astype(jnp.float32)
        out_copy(i, slot).start()
        nxt = i + _NBUF
        if nxt < nchunk:
            in_copy(nxt, slot).start()

    for i in range(max(nchunk - _NBUF, 0), nchunk):
        out_copy(i, i % _NBUF).wait()


@jax.jit
def _lsh(x, random_vectors):
    n = x.shape[0]
    return pl.pallas_call(
        _lsh_manual,
        in_specs=[
            pl.BlockSpec(memory_space=pl.ANY),
            pl.BlockSpec(memory_space=pltpu.VMEM),
        ],
        out_specs=pl.BlockSpec(memory_space=pl.ANY),
        out_shape=jax.ShapeDtypeStruct((n, _N_HASHES), jnp.float32),
        scratch_shapes=[
            pltpu.VMEM((_NBUF * _CHUNK, _DIM), jnp.float32),
            pltpu.VMEM((_NBUF * _CHUNK, _N_HASHES), jnp.float32),
            pltpu.SemaphoreType.DMA((_NBUF,)),
            pltpu.SemaphoreType.DMA((_NBUF,)),
        ],
    )(x, random_vectors)


def kernel(x, random_vectors):
    return _lsh(x, random_vectors)


# speedup vs baseline: 1.0144x; 1.0144x over previous
import functools

import jax
import jax.numpy as jnp
from jax.experimental import pallas as pl
from jax.experimental.pallas import tpu as pltpu

_DIM = 512
_N_HASHES = 256
_BANDWIDTH = 4.0
_N_BUCKETS = 1024

_CHUNK = 2048
_NBUF = 8


def _lsh_manual(x_hbm, rv_ref, out_hbm, x_buf, out_buf, in_sems, out_sems):
    n = x_hbm.shape[0]
    nchunk = n // _CHUNK

    def in_copy(i, slot):
        return pltpu.make_async_copy(
            x_hbm.at[pl.ds(i * _CHUNK, _CHUNK), :],
            x_buf.at[pl.ds(slot * _CHUNK, _CHUNK), :],
            in_sems.at[slot],
        )

    def out_copy(i, slot):
        return pltpu.make_async_copy(
            out_buf.at[pl.ds(slot * _CHUNK, _CHUNK), :],
            out_hbm.at[pl.ds(i * _CHUNK, _CHUNK), :],
            out_sems.at[slot],
        )

    for b in range(min(_NBUF, nchunk)):
        in_copy(b, b).start()

    for i in range(nchunk):
        slot = i % _NBUF
        in_copy(i, slot).wait()
        if i >= _NBUF:
            out_copy(i - _NBUF, slot).wait()
        xs = x_buf[pl.ds(slot * _CHUNK, _CHUNK), :]
        proj = jnp.dot(xs, rv_ref[...], preferred_element_type=jnp.float32)
        buckets = jnp.floor(proj * (1.0 / _BANDWIDTH)).astype(jnp.int32) & (
            _N_BUCKETS - 1
        )
        out_buf[pl.ds(slot * _CHUNK, _CHUNK), :] = buckets.astype(jnp.float32)
        out_copy(i, slot).start()
        nxt = i + _NBUF
        if nxt < nchunk:
            in_copy(nxt, slot).start()

    for i in range(max(nchunk - _NBUF, 0), nchunk):
        out_copy(i, i % _NBUF).wait()


@jax.jit
def _lsh(x, random_vectors):
    n = x.shape[0]
    return pl.pallas_call(
        _lsh_manual,
        in_specs=[
            pl.BlockSpec(memory_space=pl.ANY),
            pl.BlockSpec(memory_space=pltpu.VMEM),
        ],
        out_specs=pl.BlockSpec(memory_space=pl.ANY),
        out_shape=jax.ShapeDtypeStruct((n, _N_HASHES), jnp.float32),
        scratch_shapes=[
            pltpu.VMEM((_NBUF * _CHUNK, _DIM), jnp.float32),
            pltpu.VMEM((_NBUF * _CHUNK, _N_HASHES), jnp.float32),
            pltpu.SemaphoreType.DMA((_NBUF,)),
            pltpu.SemaphoreType.DMA((_NBUF,)),
        ],
    )(x, random_vectors)


def kernel(x, random_vectors):
    return _lsh(x, random_vectors)
